# Initial kernel scaffold; baseline (speedup 1.0000x reference)
import jax
import jax.numpy as jnp
from jax.experimental import pallas as pl


def kernel(x, edge_index, ctx_nodes, W1, b1, g1, be1, W2, b2, g2, be2, Wc1, bc1, Wc2, bc2, Wh1, bh1, Wh2, bh2):
    raise NotImplementedError("write your pallas kernel here")


# speedup vs baseline: 16.0742x; 16.0742x over previous
import functools

import jax
import jax.numpy as jnp
from jax import lax
from jax.experimental import pallas as pl
from jax.experimental.pallas import tpu as pltpu
from jax.experimental.pallas import tpu_sc as plsc

N = 10000
E = 320000
D = 128
H = 128
N2 = 10240
RPT = 640
EB = 128
NBLK = E // EB

_mesh = plsc.VectorSubcoreMesh(core_axis_name="c", subcore_axis_name="s")


def _fill(ref, val, n):
    def body(i, _):
        ref[pl.ds(i * 16, 16)] = jnp.full((16,), val, jnp.float32)
        return 0
    lax.fori_loop(0, n // 16, body, 0)


@functools.partial(
    pl.kernel,
    out_type=jax.ShapeDtypeStruct((2 * N2,), jnp.float32),
    mesh=_mesh,
    scratch_types=[
        pltpu.VMEM_SHARED((N2,), jnp.float32),
        pltpu.VMEM((EB,), jnp.int32),
        pltpu.VMEM((EB,), jnp.float32),
        pltpu.VMEM((RPT,), jnp.float32),
    ],
)
def _sc_degree(dst_hbm, out_hbm, degS, idxb, ones, zb):
    c = lax.axis_index("c")
    s = lax.axis_index("s")
    wid = c * 16 + s
    _fill(zb, 0.0, RPT)
    _fill(ones, 1.0, EB)
    pltpu.sync_copy(zb, degS.at[pl.ds(s * RPT, RPT)])
    plsc.subcore_barrier()

    nrows = (NBLK - wid + 31) // 32

    def body(i, _):
        j = wid + i * 32
        pltpu.sync_copy(dst_hbm.at[pl.ds(j * EB, EB)], idxb)
        pltpu.sync_copy(ones, degS.at[idxb], add=True)
        return 0
    lax.fori_loop(0, nrows, body, 0)

    plsc.subcore_barrier()

    @pl.when(s == 0)
    def _():
        pltpu.sync_copy(degS, out_hbm.at[pl.ds(c * N2, N2)])


@functools.partial(
    pl.kernel,
    out_type=[jax.ShapeDtypeStruct((N2, D), jnp.float32),
              jax.ShapeDtypeStruct((N2, D), jnp.float32)],
    mesh=_mesh,
    scratch_types=[
        pltpu.VMEM_SHARED((N2, D), jnp.float32),
        pltpu.VMEM((EB,), jnp.int32),
        pltpu.VMEM((EB,), jnp.int32),
        pltpu.VMEM((EB, D), jnp.float32),
        pltpu.VMEM((64, D), jnp.float32),
        pltpu.SemaphoreType.DMA,
    ],
)
def _sc_agg(u_hbm, src_hbm, dst_hbm, outa, outb, agg, sib, dib, gb, zb, sem):
    c = lax.axis_index("c")
    s = lax.axis_index("s")
    wid = c * 16 + s

    def zfill(i, _):
        zb[pl.ds(i * 16, 16), :] = jnp.zeros((16, D), jnp.float32)
        return 0
    lax.fori_loop(0, 4, zfill, 0)

    def zcopy(k, _):
        pltpu.sync_copy(zb, agg.at[pl.ds(s * RPT + k * 64, 64)])
        return 0
    lax.fori_loop(0, RPT // 64, zcopy, 0)
    plsc.subcore_barrier()

    nrows = (NBLK - wid + 31) // 32

    def body(i, _):
        j = wid + i * 32
        pltpu.sync_copy(src_hbm.at[pl.ds(j * EB, EB)], sib)
        pltpu.sync_copy(dst_hbm.at[pl.ds(j * EB, EB)], dib)
        pltpu.async_copy(u_hbm.at[sib], gb, sem).wait()
        pltpu.sync_copy(gb, agg.at[dib], add=True)
        return 0
    lax.fori_loop(0, nrows, body, 0)

    plsc.subcore_barrier()

    @pl.when(c == 0)
    def _():
        pltpu.sync_copy(agg.at[pl.ds(s * RPT, RPT)], outa.at[pl.ds(s * RPT, RPT)])

    @pl.when(c == 1)
    def _():
        pltpu.sync_copy(agg.at[pl.ds(s * RPT, RPT)], outb.at[pl.ds(s * RPT, RPT)])


_BLK = 1280
_GRID = N2 // _BLK


def _row_spec(w):
    return pl.BlockSpec((_BLK, w), lambda i: (i, 0))


def _full_spec(shape):
    nd = len(shape)
    return pl.BlockSpec(shape, lambda i: (0,) * nd)


def _m1_body(x_ref, dega_ref, degb_ref, w1_ref, ctx_ref, wc1_ref, bc1_ref,
             wc2_ref, bc2_ref, wh1b_ref, u1_ref, cpart_ref, dinv_ref):
    dinv = lax.rsqrt(dega_ref[...] + degb_ref[...] + 1.0)
    hw = jnp.dot(x_ref[...], w1_ref[...], preferred_element_type=jnp.float32)
    u1_ref[...] = hw * dinv
    dinv_ref[...] = dinv
    c1 = jax.nn.relu(jnp.dot(ctx_ref[...], wc1_ref[...],
                             preferred_element_type=jnp.float32) + bc1_ref[...])
    c2 = jnp.dot(c1, wc2_ref[...], preferred_element_type=jnp.float32) + bc2_ref[...]
    cpart_ref[...] = jnp.dot(c2, wh1b_ref[...], preferred_element_type=jnp.float32)


def _ln(t, g, b):
    mu = jnp.mean(t, axis=1, keepdims=True)
    var = jnp.mean((t - mu) * (t - mu), axis=1, keepdims=True)
    return g * (t - mu) * lax.rsqrt(var + 1e-5) + b


def _m2_body(agga_ref, aggb_ref, u_ref, dinv_ref, b_ref, g_ref,
             be_ref, w_ref, u2_ref):
    dinv = dinv_ref[...]
    t = (agga_ref[...] + aggb_ref[...] + u_ref[...]) * dinv + b_ref[...]
    h = jax.nn.relu(_ln(t, g_ref[...], be_ref[...]))
    u2_ref[...] = jnp.dot(h, w_ref[...], preferred_element_type=jnp.float32) * dinv


def _m3_body(agga_ref, aggb_ref, u_ref, dinv_ref, b_ref, g_ref,
             be_ref, cpart_ref, wh1t_ref, bh1_ref, wh2_ref, bh2_ref, out_ref):
    t = (agga_ref[...] + aggb_ref[...] + u_ref[...]) * dinv_ref[...] + b_ref[...]
    h2 = jax.nn.relu(_ln(t, g_ref[...], be_ref[...]))
    a = jax.nn.relu(jnp.dot(h2, wh1t_ref[...], preferred_element_type=jnp.float32)
                    + cpart_ref[...] + bh1_ref[...])
    out_ref[...] = jnp.dot(a, wh2_ref[...],
                           preferred_element_type=jnp.float32) + bh2_ref[...]


def kernel(x, edge_index, ctx_nodes, W1, b1, g1, be1, W2, b2, g2, be2,
           Wc1, bc1, Wc2, bc2, Wh1, bh1, Wh2, bh2):
    src = edge_index[0]
    dst = edge_index[1]
    x_p = jnp.pad(x, ((0, N2 - N), (0, 0)))
    ctx_p = jnp.pad(ctx_nodes, ((0, N2 - N), (0, 0)))

    deg2 = _sc_degree(dst)
    dega = deg2[:N2].reshape(N2, 1)
    degb = deg2[N2:].reshape(N2, 1)

    u1, cpart, dinv = pl.pallas_call(
        _m1_body,
        grid=(_GRID,),
        in_specs=[
            _row_spec(D), _row_spec(1), _row_spec(1), _full_spec((D, H)),
            _row_spec(16), _full_spec((16, 16)), _full_spec((1, 16)),
            _full_spec((16, H)), _full_spec((1, H)), _full_spec((H, H)),
        ],
        out_specs=[_row_spec(H), _row_spec(H), _row_spec(1)],
        out_shape=[
            jax.ShapeDtypeStruct((N2, H), jnp.float32),
            jax.ShapeDtypeStruct((N2, H), jnp.float32),
            jax.ShapeDtypeStruct((N2, 1), jnp.float32),
        ],
    )(x_p, dega, degb, W1, ctx_p, Wc1, bc1.reshape(1, 16), Wc2,
      bc2.reshape(1, H), Wh1[H:])

    agg1a, agg1b = _sc_agg(u1, src, dst)

    u2 = pl.pallas_call(
        _m2_body,
        grid=(_GRID,),
        in_specs=[
            _row_spec(H), _row_spec(H), _row_spec(H), _row_spec(1),
            _full_spec((1, H)), _full_spec((1, H)), _full_spec((1, H)),
            _full_spec((H, H)),
        ],
        out_specs=_row_spec(H),
        out_shape=jax.ShapeDtypeStruct((N2, H), jnp.float32),
    )(agg1a, agg1b, u1, dinv, b1.reshape(1, H), g1.reshape(1, H),
      be1.reshape(1, H), W2)

    agg2a, agg2b = _sc_agg(u2, src, dst)

    logits = pl.pallas_call(
        _m3_body,
        grid=(_GRID,),
        in_specs=[
            _row_spec(H), _row_spec(H), _row_spec(H), _row_spec(1),
            _full_spec((1, H)), _full_spec((1, H)), _full_spec((1, H)),
            _row_spec(H), _full_spec((H, H)), _full_spec((1, H)),
            _full_spec((H, 8)), _full_spec((1, 8)),
        ],
        out_specs=_row_spec(8),
        out_shape=jax.ShapeDtypeStruct((N2, 8), jnp.float32),
    )(agg2a, agg2b, u2, dinv, b2.reshape(1, H), g2.reshape(1, H),
      be2.reshape(1, H), cpart, Wh1[:H], bh1.reshape(1, H), Wh2,
      bh2.reshape(1, 8))

    return logits[:N]


# speedup vs baseline: 27.5249x; 1.7124x over previous
import functools

import jax
import jax.numpy as jnp
from jax import lax
from jax.experimental import pallas as pl
from jax.experimental.pallas import tpu as pltpu
from jax.experimental.pallas import tpu_sc as plsc

N = 10000
E = 320000
D = 128
H = 128
N2 = 10240
RPT = 640
EB = 64
BPW = 160
NBLK = 32 * BPW
EPAD = NBLK * EB - E
NBUF = 2
PD = 1
SB = 32
NSB = BPW // SB

_mesh = plsc.VectorSubcoreMesh(core_axis_name="c", subcore_axis_name="s")


def _fill(ref, val, n):
    def body(i, _):
        ref[pl.ds(i * 16, 16)] = jnp.full((16,), val, jnp.float32)
        return 0
    lax.fori_loop(0, n // 16, body, 0)


@functools.partial(
    pl.kernel,
    out_type=jax.ShapeDtypeStruct((2 * N2,), jnp.float32),
    mesh=_mesh,
    scratch_types=[
        pltpu.VMEM_SHARED((N2,), jnp.float32),
        pltpu.VMEM((BPW, EB), jnp.int32),
        pltpu.VMEM((EB,), jnp.float32),
        pltpu.VMEM((RPT,), jnp.float32),
        pltpu.SemaphoreType.DMA,
    ],
)
def _sc_degree(dst_hbm, out_hbm, degS, diball, ones, zb, ssem):
    c = lax.axis_index("c")
    s = lax.axis_index("s")
    wid = c * 16 + s
    _fill(zb, 0.0, RPT)
    _fill(ones, 1.0, EB)
    pltpu.sync_copy(dst_hbm.at[pl.ds(wid * BPW, BPW)], diball)
    pltpu.sync_copy(zb, degS.at[pl.ds(s * RPT, RPT)])
    plsc.subcore_barrier()

    def fire(j, _):
        pltpu.async_copy(ones, degS.at[diball.at[j]], ssem, add=True)
        return 0
    lax.fori_loop(0, BPW, fire, 0)

    def drain(j, _):
        pltpu.make_async_copy(ones, degS.at[diball.at[0]], ssem).wait()
        return 0
    lax.fori_loop(0, BPW, drain, 0)

    plsc.subcore_barrier()

    @pl.when(s == 0)
    def _():
        pltpu.sync_copy(degS, out_hbm.at[pl.ds(c * N2, N2)])


@functools.partial(
    pl.kernel,
    out_type=[jax.ShapeDtypeStruct((N2, D), jnp.float32),
              jax.ShapeDtypeStruct((N2, D), jnp.float32)],
    mesh=_mesh,
    scratch_types=[
        pltpu.VMEM_SHARED((N2, D), jnp.float32),
        pltpu.VMEM((2, SB, EB), jnp.int32),
        pltpu.VMEM((2, SB, EB), jnp.int32),
        pltpu.VMEM((NBUF, EB, D), jnp.float32),
        pltpu.VMEM((16, D), jnp.float32),
        pltpu.SemaphoreType.DMA((NBUF,)),
        pltpu.SemaphoreType.DMA((NBUF,)),
        pltpu.SemaphoreType.DMA,
        pltpu.SemaphoreType.DMA,
    ],
)
def _sc_agg(u_hbm, src_hbm, dst_hbm, outa, outb,
            agg, sib, dib, gbuf, zb, gsem, ssem, psem, qsem):
    c = lax.axis_index("c")
    s = lax.axis_index("s")
    wid = c * 16 + s
    start = wid * BPW

    zb[...] = jnp.zeros((16, D), jnp.float32)

    def zcopy(k, _):
        pltpu.sync_copy(zb, agg.at[pl.ds(s * RPT + k * 16, 16)])
        return 0
    lax.fori_loop(0, RPT // 16, zcopy, 0)

    pltpu.async_copy(src_hbm.at[pl.ds(start, SB)], sib.at[0], psem)
    pltpu.async_copy(dst_hbm.at[pl.ds(start, SB)], dib.at[0], qsem)
    plsc.subcore_barrier()

    def step(j, _):
        q = lax.div(j, SB)
        r = lax.rem(j, SB)
        sl = lax.rem(q, 2)
        b = lax.rem(j, NBUF)
        jp = j - PD
        bp = lax.rem(jp + NBUF, NBUF)
        qp = lax.div(jp + SB, SB) - 1
        rp = jp - qp * SB
        slp = lax.rem(qp + 2, 2)

        @pl.when(j >= NBUF)
        def _():
            pltpu.make_async_copy(gbuf.at[b], agg.at[dib.at[0, 0]],
                                  ssem.at[b]).wait()

        @pl.when((r == 0) & (j < BPW))
        def _():
            pltpu.make_async_copy(src_hbm.at[pl.ds(start, SB)],
                                  sib.at[0], psem).wait()
            pltpu.make_async_copy(dst_hbm.at[pl.ds(start, SB)],
                                  dib.at[0], qsem).wait()

        @pl.when((r == NBUF) & (j + SB < BPW))
        def _():
            off = start + (q + 1) * SB
            pltpu.async_copy(src_hbm.at[pl.ds(off, SB)],
                             sib.at[lax.rem(q + 1, 2)], psem)
            pltpu.async_copy(dst_hbm.at[pl.ds(off, SB)],
                             dib.at[lax.rem(q + 1, 2)], qsem)

        @pl.when(j < BPW)
        def _():
            pltpu.async_copy(u_hbm.at[sib.at[sl, r]], gbuf.at[b], gsem.at[b])

        @pl.when((jp >= 0) & (jp < BPW))
        def _():
            pltpu.make_async_copy(u_hbm.at[sib.at[0, 0]],
                                  gbuf.at[bp], gsem.at[bp]).wait()
            pltpu.async_copy(gbuf.at[bp], agg.at[dib.at[slp, rp]],
                             ssem.at[bp], add=True)
        return 0
    lax.fori_loop(0, BPW + NBUF, step, 0)

    plsc.subcore_barrier()

    @pl.when(c == 0)
    def _():
        pltpu.sync_copy(agg.at[pl.ds(s * RPT, RPT)], outa.at[pl.ds(s * RPT, RPT)])

    @pl.when(c == 1)
    def _():
        pltpu.sync_copy(agg.at[pl.ds(s * RPT, RPT)], outb.at[pl.ds(s * RPT, RPT)])


_BLK = 1280
_GRID = N2 // _BLK


def _row_spec(w):
    return pl.BlockSpec((_BLK, w), lambda i: (i, 0))


def _full_spec(shape):
    nd = len(shape)
    return pl.BlockSpec(shape, lambda i: (0,) * nd)


def _m1_body(x_ref, dega_ref, degb_ref, w1_ref, ctx_ref, wc1_ref, bc1_ref,
             wc2_ref, bc2_ref, wh1b_ref, u1_ref, cpart_ref, dinv_ref):
    dinv = lax.rsqrt(dega_ref[...] + degb_ref[...] + 1.0)
    hw = jnp.dot(x_ref[...], w1_ref[...], preferred_element_type=jnp.float32)
    u1_ref[...] = hw * dinv
    dinv_ref[...] = dinv
    c1 = jax.nn.relu(jnp.dot(ctx_ref[...], wc1_ref[...],
                             preferred_element_type=jnp.float32) + bc1_ref[...])
    c2 = jnp.dot(c1, wc2_ref[...], preferred_element_type=jnp.float32) + bc2_ref[...]
    cpart_ref[...] = jnp.dot(c2, wh1b_ref[...], preferred_element_type=jnp.float32)


def _ln(t, g, b):
    mu = jnp.mean(t, axis=1, keepdims=True)
    var = jnp.mean((t - mu) * (t - mu), axis=1, keepdims=True)
    return g * (t - mu) * lax.rsqrt(var + 1e-5) + b


def _m2_body(agga_ref, aggb_ref, u_ref, dinv_ref, b_ref, g_ref,
             be_ref, w_ref, u2_ref):
    dinv = dinv_ref[...]
    t = (agga_ref[...] + aggb_ref[...] + u_ref[...]) * dinv + b_ref[...]
    h = jax.nn.relu(_ln(t, g_ref[...], be_ref[...]))
    u2_ref[...] = jnp.dot(h, w_ref[...], preferred_element_type=jnp.float32) * dinv


def _m3_body(agga_ref, aggb_ref, u_ref, dinv_ref, b_ref, g_ref,
             be_ref, cpart_ref, wh1t_ref, bh1_ref, wh2_ref, bh2_ref, out_ref):
    t = (agga_ref[...] + aggb_ref[...] + u_ref[...]) * dinv_ref[...] + b_ref[...]
    h2 = jax.nn.relu(_ln(t, g_ref[...], be_ref[...]))
    a = jax.nn.relu(jnp.dot(h2, wh1t_ref[...], preferred_element_type=jnp.float32)
                    + cpart_ref[...] + bh1_ref[...])
    out_ref[...] = jnp.dot(a, wh2_ref[...],
                           preferred_element_type=jnp.float32) + bh2_ref[...]


def kernel(x, edge_index, ctx_nodes, W1, b1, g1, be1, W2, b2, g2, be2,
           Wc1, bc1, Wc2, bc2, Wh1, bh1, Wh2, bh2):
    pidx = jnp.arange(EPAD, dtype=jnp.int32)
    src_p = jnp.concatenate([edge_index[0], pidx % N])
    dst_p = jnp.concatenate([edge_index[1], N + pidx % (N2 - N)])
    src2d = src_p.reshape(NBLK, EB)
    dst2d = dst_p.reshape(NBLK, EB)
    x_p = jnp.pad(x, ((0, N2 - N), (0, 0)))
    ctx_p = jnp.pad(ctx_nodes, ((0, N2 - N), (0, 0)))

    deg2 = _sc_degree(dst2d)
    dega = deg2[:N2].reshape(N2, 1)
    degb = deg2[N2:].reshape(N2, 1)

    u1, cpart, dinv = pl.pallas_call(
        _m1_body,
        grid=(_GRID,),
        in_specs=[
            _row_spec(D), _row_spec(1), _row_spec(1), _full_spec((D, H)),
            _row_spec(16), _full_spec((16, 16)), _full_spec((1, 16)),
            _full_spec((16, H)), _full_spec((1, H)), _full_spec((H, H)),
        ],
        out_specs=[_row_spec(H), _row_spec(H), _row_spec(1)],
        out_shape=[
            jax.ShapeDtypeStruct((N2, H), jnp.float32),
            jax.ShapeDtypeStruct((N2, H), jnp.float32),
            jax.ShapeDtypeStruct((N2, 1), jnp.float32),
        ],
    )(x_p, dega, degb, W1, ctx_p, Wc1, bc1.reshape(1, 16), Wc2,
      bc2.reshape(1, H), Wh1[H:])

    agg1a, agg1b = _sc_agg(u1, src2d, dst2d)

    u2 = pl.pallas_call(
        _m2_body,
        grid=(_GRID,),
        in_specs=[
            _row_spec(H), _row_spec(H), _row_spec(H), _row_spec(1),
            _full_spec((1, H)), _full_spec((1, H)), _full_spec((1, H)),
            _full_spec((H, H)),
        ],
        out_specs=_row_spec(H),
        out_shape=jax.ShapeDtypeStruct((N2, H), jnp.float32),
    )(agg1a, agg1b, u1, dinv, b1.reshape(1, H), g1.reshape(1, H),
      be1.reshape(1, H), W2)

    agg2a, agg2b = _sc_agg(u2, src2d, dst2d)

    logits = pl.pallas_call(
        _m3_body,
        grid=(_GRID,),
        in_specs=[
            _row_spec(H), _row_spec(H), _row_spec(H), _row_spec(1),
            _full_spec((1, H)), _full_spec((1, H)), _full_spec((1, H)),
            _row_spec(H), _full_spec((H, H)), _full_spec((1, H)),
            _full_spec((H, 8)), _full_spec((1, 8)),
        ],
        out_specs=_row_spec(8),
        out_shape=jax.ShapeDtypeStruct((N2, 8), jnp.float32),
    )(agg2a, agg2b, u2, dinv, b2.reshape(1, H), g2.reshape(1, H),
      be2.reshape(1, H), cpart, Wh1[:H], bh1.reshape(1, H), Wh2,
      bh2.reshape(1, 8))

    return logits[:N]


# speedup vs baseline: 28.2436x; 1.0261x over previous
import functools

import jax
import jax.numpy as jnp
from jax import lax
from jax.experimental import pallas as pl
from jax.experimental.pallas import tpu as pltpu
from jax.experimental.pallas import tpu_sc as plsc

N = 10000
E = 320000
D = 128
H = 128
N2 = 10240
RPT = 640
EB = 64
BPW = 160
NBLK = 32 * BPW
EPAD = NBLK * EB - E
NBUF = 2
PD = 1
SB = 32
NSB = BPW // SB

_mesh = plsc.VectorSubcoreMesh(core_axis_name="c", subcore_axis_name="s")


def _fill(ref, val, n):
    def body(i, _):
        ref[pl.ds(i * 16, 16)] = jnp.full((16,), val, jnp.float32)
        return 0
    lax.fori_loop(0, n // 16, body, 0)


@functools.partial(
    pl.kernel,
    out_type=jax.ShapeDtypeStruct((2 * N2,), jnp.float32),
    mesh=_mesh,
    scratch_types=[
        pltpu.VMEM_SHARED((N2,), jnp.float32),
        pltpu.VMEM((BPW, EB), jnp.int32),
        pltpu.VMEM((EB,), jnp.float32),
        pltpu.VMEM((RPT,), jnp.float32),
        pltpu.SemaphoreType.DMA,
    ],
)
def _sc_degree(dst_hbm, out_hbm, degS, diball, ones, zb, ssem):
    c = lax.axis_index("c")
    s = lax.axis_index("s")
    wid = c * 16 + s
    _fill(zb, 0.0, RPT)
    _fill(ones, 1.0, EB)
    pltpu.sync_copy(dst_hbm.at[pl.ds(wid * BPW, BPW)], diball)
    pltpu.sync_copy(zb, degS.at[pl.ds(s * RPT, RPT)])
    plsc.subcore_barrier()

    def fire(j, _):
        pltpu.async_copy(ones, degS.at[diball.at[j]], ssem, add=True)
        return 0
    lax.fori_loop(0, BPW, fire, 0)

    def drain(j, _):
        pltpu.make_async_copy(ones, degS.at[diball.at[0]], ssem).wait()
        return 0
    lax.fori_loop(0, BPW, drain, 0)

    plsc.subcore_barrier()

    @pl.when(s == 0)
    def _():
        pltpu.sync_copy(degS, out_hbm.at[pl.ds(c * N2, N2)])


@functools.partial(
    pl.kernel,
    out_type=[jax.ShapeDtypeStruct((N2, D), jnp.float32),
              jax.ShapeDtypeStruct((N2, D), jnp.float32)],
    mesh=_mesh,
    scratch_types=[
        pltpu.VMEM_SHARED((N2, D), jnp.float32),
        pltpu.VMEM((2, SB, EB), jnp.int32),
        pltpu.VMEM((2, SB, EB), jnp.int32),
        pltpu.VMEM((NBUF, EB, D), jnp.float32),
        pltpu.VMEM((16, D), jnp.float32),
        pltpu.SemaphoreType.DMA((NBUF,)),
        pltpu.SemaphoreType.DMA((NBUF,)),
        pltpu.SemaphoreType.DMA,
        pltpu.SemaphoreType.DMA,
    ],
)
def _sc_agg(u_hbm, src_hbm, dst_hbm, outa, outb,
            agg, sib, dib, gbuf, zb, gsem, ssem, psem, qsem):
    c = lax.axis_index("c")
    s = lax.axis_index("s")
    wid = c * 16 + s
    start = wid * BPW

    zb[...] = jnp.zeros((16, D), jnp.float32)

    def zcopy(k, _):
        pltpu.sync_copy(zb, agg.at[pl.ds(s * RPT + k * 16, 16)])
        return 0
    lax.fori_loop(0, RPT // 16, zcopy, 0)

    pltpu.async_copy(src_hbm.at[pl.ds(start, SB)], sib.at[0], psem)
    pltpu.async_copy(dst_hbm.at[pl.ds(start, SB)], dib.at[0], qsem)
    plsc.subcore_barrier()

    def step(j, _):
        q = lax.div(j, SB)
        r = lax.rem(j, SB)
        sl = lax.rem(q, 2)
        b = lax.rem(j, NBUF)
        jp = j - PD
        bp = lax.rem(jp + NBUF, NBUF)
        qp = lax.div(jp + SB, SB) - 1
        rp = jp - qp * SB
        slp = lax.rem(qp + 2, 2)

        @pl.when(j >= NBUF)
        def _():
            pltpu.make_async_copy(gbuf.at[b], agg.at[dib.at[0, 0]],
                                  ssem.at[b]).wait()

        @pl.when((r == 0) & (j < BPW))
        def _():
            pltpu.make_async_copy(src_hbm.at[pl.ds(start, SB)],
                                  sib.at[0], psem).wait()
            pltpu.make_async_copy(dst_hbm.at[pl.ds(start, SB)],
                                  dib.at[0], qsem).wait()

        @pl.when((r == NBUF) & (j + SB < BPW))
        def _():
            off = start + (q + 1) * SB
            pltpu.async_copy(src_hbm.at[pl.ds(off, SB)],
                             sib.at[lax.rem(q + 1, 2)], psem)
            pltpu.async_copy(dst_hbm.at[pl.ds(off, SB)],
                             dib.at[lax.rem(q + 1, 2)], qsem)

        @pl.when(j < BPW)
        def _():
            pltpu.async_copy(u_hbm.at[sib.at[sl, r]], gbuf.at[b], gsem.at[b])

        @pl.when((jp >= 0) & (jp < BPW))
        def _():
            pltpu.make_async_copy(u_hbm.at[sib.at[0, 0]],
                                  gbuf.at[bp], gsem.at[bp]).wait()
            pltpu.async_copy(gbuf.at[bp], agg.at[dib.at[slp, rp]],
                             ssem.at[bp], add=True)
        return 0
    lax.fori_loop(0, BPW + NBUF, step, 0)

    plsc.subcore_barrier()

    @pl.when(c == 0)
    def _():
        pltpu.sync_copy(agg.at[pl.ds(s * RPT, RPT)], outa.at[pl.ds(s * RPT, RPT)])

    @pl.when(c == 1)
    def _():
        pltpu.sync_copy(agg.at[pl.ds(s * RPT, RPT)], outb.at[pl.ds(s * RPT, RPT)])


_BLK = 1280
_GRID = N2 // _BLK


def _row_spec(w):
    return pl.BlockSpec((_BLK, w), lambda i: (i, 0))


def _full_spec(shape):
    nd = len(shape)
    return pl.BlockSpec(shape, lambda i: (0,) * nd)


def _k1_body(x_ref, dega_ref, degb_ref, v1_ref, dinv_ref):
    dinv = lax.rsqrt(dega_ref[...] + degb_ref[...] + 1.0)
    v1_ref[...] = x_ref[...] * dinv
    dinv_ref[...] = dinv


def _kc_body(ctx_ref, wc1_ref, bc1_ref, wc2_ref, bc2_ref, wh1b_ref,
             cpart_ref):
    c1 = jax.nn.relu(jnp.dot(ctx_ref[...], wc1_ref[...],
                             preferred_element_type=jnp.float32) + bc1_ref[...])
    c2 = jnp.dot(c1, wc2_ref[...], preferred_element_type=jnp.float32) + bc2_ref[...]
    cpart_ref[...] = jnp.dot(c2, wh1b_ref[...], preferred_element_type=jnp.float32)


def _ln(t, g, b):
    mu = jnp.mean(t, axis=1, keepdims=True)
    var = jnp.mean((t - mu) * (t - mu), axis=1, keepdims=True)
    return g * (t - mu) * lax.rsqrt(var + 1e-5) + b


def _k2_body(agga_ref, aggb_ref, v_ref, dinv_ref, w_ref, b_ref, g_ref,
             be_ref, v2_ref):
    dinv = dinv_ref[...]
    s = (agga_ref[...] + aggb_ref[...] + v_ref[...]) * dinv
    t = jnp.dot(s, w_ref[...], preferred_element_type=jnp.float32) + b_ref[...]
    h = jax.nn.relu(_ln(t, g_ref[...], be_ref[...]))
    v2_ref[...] = h * dinv


def _k3_body(agga_ref, aggb_ref, v_ref, dinv_ref, w_ref, b_ref, g_ref,
             be_ref, cpart_ref, wh1t_ref, bh1_ref, wh2_ref, bh2_ref, out_ref):
    s = (agga_ref[...] + aggb_ref[...] + v_ref[...]) * dinv_ref[...]
    t = jnp.dot(s, w_ref[...], preferred_element_type=jnp.float32) + b_ref[...]
    h2 = jax.nn.relu(_ln(t, g_ref[...], be_ref[...]))
    a = jax.nn.relu(jnp.dot(h2, wh1t_ref[...], preferred_element_type=jnp.float32)
                    + cpart_ref[...] + bh1_ref[...])
    out_ref[...] = jnp.dot(a, wh2_ref[...],
                           preferred_element_type=jnp.float32) + bh2_ref[...]


def kernel(x, edge_index, ctx_nodes, W1, b1, g1, be1, W2, b2, g2, be2,
           Wc1, bc1, Wc2, bc2, Wh1, bh1, Wh2, bh2):
    pidx = jnp.arange(EPAD, dtype=jnp.int32)
    src_p = jnp.concatenate([edge_index[0], pidx % N])
    dst_p = jnp.concatenate([edge_index[1], N + pidx % (N2 - N)])
    src2d = src_p.reshape(NBLK, EB)
    dst2d = dst_p.reshape(NBLK, EB)
    x_p = jnp.pad(x, ((0, N2 - N), (0, 0)))
    ctx_p = jnp.pad(ctx_nodes, ((0, N2 - N), (0, 0)))

    deg2 = _sc_degree(dst2d)
    dega = deg2[:N2].reshape(N2, 1)
    degb = deg2[N2:].reshape(N2, 1)

    v1, dinv = pl.pallas_call(
        _k1_body,
        grid=(_GRID,),
        in_specs=[_row_spec(D), _row_spec(1), _row_spec(1)],
        out_specs=[_row_spec(H), _row_spec(1)],
        out_shape=[
            jax.ShapeDtypeStruct((N2, H), jnp.float32),
            jax.ShapeDtypeStruct((N2, 1), jnp.float32),
        ],
    )(x_p, dega, degb)

    cpart = pl.pallas_call(
        _kc_body,
        grid=(_GRID,),
        in_specs=[
            _row_spec(16), _full_spec((16, 16)), _full_spec((1, 16)),
            _full_spec((16, H)), _full_spec((1, H)), _full_spec((H, H)),
        ],
        out_specs=_row_spec(H),
        out_shape=jax.ShapeDtypeStruct((N2, H), jnp.float32),
    )(ctx_p, Wc1, bc1.reshape(1, 16), Wc2, bc2.reshape(1, H), Wh1[H:])

    agg1a, agg1b = _sc_agg(v1, src2d, dst2d)

    v2 = pl.pallas_call(
        _k2_body,
        grid=(_GRID,),
        in_specs=[
            _row_spec(H), _row_spec(H), _row_spec(H), _row_spec(1),
            _full_spec((D, H)), _full_spec((1, H)), _full_spec((1, H)),
            _full_spec((1, H)),
        ],
        out_specs=_row_spec(H),
        out_shape=jax.ShapeDtypeStruct((N2, H), jnp.float32),
    )(agg1a, agg1b, v1, dinv, W1, b1.reshape(1, H), g1.reshape(1, H),
      be1.reshape(1, H))

    agg2a, agg2b = _sc_agg(v2, src2d, dst2d)

    logits = pl.pallas_call(
        _k3_body,
        grid=(_GRID,),
        in_specs=[
            _row_spec(H), _row_spec(H), _row_spec(H), _row_spec(1),
            _full_spec((H, H)), _full_spec((1, H)), _full_spec((1, H)),
            _full_spec((1, H)),
            _row_spec(H), _full_spec((H, H)), _full_spec((1, H)),
            _full_spec((H, 8)), _full_spec((1, 8)),
        ],
        out_specs=_row_spec(8),
        out_shape=jax.ShapeDtypeStruct((N2, 8), jnp.float32),
    )(agg2a, agg2b, v2, dinv, W2, b2.reshape(1, H), g2.reshape(1, H),
      be2.reshape(1, H), cpart, Wh1[:H], bh1.reshape(1, H), Wh2,
      bh2.reshape(1, 8))

    return logits[:N]
